# Initial kernel scaffold; baseline (speedup 1.0000x reference)
import jax
import jax.numpy as jnp
from jax.experimental import pallas as pl


def kernel(x, edge_index, batch, W_emb, b_emb, W_a1, b_a1, W_a2, b_a2, W_head, b_head):
    raise NotImplementedError("write your pallas kernel here")


# speedup vs baseline: 55.8430x; 55.8430x over previous
import functools

import jax
import jax.numpy as jnp
from jax import lax
from jax.experimental import pallas as pl
from jax.experimental.pallas import tpu as pltpu
from jax.experimental.pallas import tpu_sc as plsc

N = 10000
NP = 10240
G = 64
TRASH = G
NSLOT = G + 1
CLEN = NSLOT * NP

E = 320000
CH = 128
NWORK = 32
EW = 10112
EPAD = NWORK * EW

_HIGH = jax.lax.Precision.HIGHEST


def _leaky(v):
    return jax.nn.leaky_relu(v, negative_slope=0.01)


def _stats_body(b_ref, nn_ref, inv_ref, sel_ref):
    bp = b_ref[...]
    gcol = lax.broadcasted_iota(jnp.int32, (G, 1), 0)
    eqg = bp == gcol
    counts = jnp.sum(eqg.astype(jnp.float32), axis=1, keepdims=True)
    nn_ref[...] = counts
    first_pos = jnp.sum((bp < gcol).astype(jnp.int32), axis=1, keepdims=True)
    tgt = jnp.where(counts > 0.0, first_pos, N - 1)
    ids = lax.broadcasted_iota(jnp.int32, (1, NP), 1)
    match = ids == tgt
    winner = jnp.max(jnp.where(match, gcol, -1), axis=0, keepdims=True)
    inv_ref[...] = jnp.where(winner >= 0, winner, TRASH)
    tgtf = tgt.astype(jnp.float32)
    eye = (lax.broadcasted_iota(jnp.int32, (G, G), 0)
           == lax.broadcasted_iota(jnp.int32, (G, G), 1)).astype(jnp.float32)
    tgt_row = lax.dot_general(tgtf, eye, (((0,), (0,)), ((), ())),
                              precision=_HIGH,
                              preferred_element_type=jnp.float32)
    mm = tgtf == tgt_row
    wio = lax.broadcasted_iota(jnp.int32, (1, G), 1)
    owner = jnp.max(jnp.where(mm, wio, -1), axis=1, keepdims=True)
    sel_ref[...] = (owner == wio).astype(jnp.float32)


_stats_call = pl.pallas_call(
    _stats_body,
    out_shape=(
        jax.ShapeDtypeStruct((G, 1), jnp.float32),
        jax.ShapeDtypeStruct((1, NP), jnp.int32),
        jax.ShapeDtypeStruct((G, G), jnp.float32),
    ),
)


def _dense_body(x_ref, we_ref, be_ref, wa1_ref, ba1_ref, wa2_ref, ba2_ref,
                emb_ref, att_ref):
    x = x_ref[...]
    emb_ref[...] = _leaky(
        lax.dot_general(x, we_ref[...], (((1,), (0,)), ((), ())),
                        precision=_HIGH, preferred_element_type=jnp.float32)
        + be_ref[...])
    a1 = _leaky(
        lax.dot_general(x, wa1_ref[...], (((1,), (0,)), ((), ())),
                        precision=_HIGH, preferred_element_type=jnp.float32)
        + ba1_ref[...])
    att_ref[...] = lax.dot_general(
        wa2_ref[...], a1, (((0,), (1,)), ((), ())),
        precision=_HIGH, preferred_element_type=jnp.float32) + ba2_ref[...]


_dense_call = pl.pallas_call(
    _dense_body,
    out_shape=(
        jax.ShapeDtypeStruct((NP, 128), jnp.float32),
        jax.ShapeDtypeStruct((1, NP), jnp.float32),
    ),
)


def _sc_counts_body(src_hbm, dst_hbm, inv_hbm, zero_hbm, out_hbm,
                    inv_v, src_v, dst_v, idx_v, ones_v, csh):
    cid = lax.axis_index("c")
    sid = lax.axis_index("s")
    wid = sid * 2 + cid
    zch = CLEN // 16

    if True:
        pltpu.sync_copy(zero_hbm.at[pl.ds(sid * zch, zch)],
                        csh.at[pl.ds(sid * zch, zch)])
        pltpu.sync_copy(inv_hbm, inv_v)
        for i in range(CH // 16):
            ones_v[pl.ds(i * 16, 16)] = jnp.full((16,), 1.0, jnp.float32)
        plsc.subcore_barrier()

        base = wid * EW

        @pl.loop(0, EW, step=CH)
        def _edge_chunk(off):
            pltpu.sync_copy(src_hbm.at[pl.ds(base + off, CH)], src_v)
            pltpu.sync_copy(dst_hbm.at[pl.ds(base + off, CH)], dst_v)
            for i in range(CH // 16):
                sl = pl.ds(i * 16, 16)
                d16 = dst_v[sl]
                s16 = src_v[sl]
                slot16 = plsc.load_gather(inv_v, [d16])
                idx_v[sl] = slot16 * NP + s16
            pltpu.sync_copy(ones_v, csh.at[idx_v], add=True)

        plsc.subcore_barrier()
        pltpu.sync_copy(csh.at[pl.ds(sid * zch, zch)],
                        out_hbm.at[cid, pl.ds(sid * zch, zch)])


_sc_counts_call = pl.kernel(
    _sc_counts_body,
    out_type=jax.ShapeDtypeStruct((2, CLEN), jnp.float32),
    mesh=plsc.VectorSubcoreMesh(core_axis_name="c", subcore_axis_name="s"),
    compiler_params=pltpu.CompilerParams(needs_layout_passes=False),
    scratch_types=[
        pltpu.VMEM((NP,), jnp.int32),
        pltpu.VMEM((CH,), jnp.int32),
        pltpu.VMEM((CH,), jnp.int32),
        pltpu.VMEM((CH,), jnp.int32),
        pltpu.VMEM((CH,), jnp.float32),
        pltpu.VMEM_SHARED((CLEN,), jnp.float32),
    ],
)


def _finish_body(c0_ref, c1_ref, att_ref, emb_ref, sel_ref, wh_ref, bh_ref,
                 out_ref):
    cs = c0_ref[...] + c1_ref[...]
    att = att_ref[...]
    pos = cs > 0.0
    m = jnp.max(jnp.where(pos, att, -jnp.inf), axis=1, keepdims=True)
    p = jnp.where(pos, cs * jnp.exp(att - m), 0.0)
    denom = jnp.sum(p, axis=1, keepdims=True)
    numer = lax.dot_general(p, emb_ref[...], (((1,), (0,)), ((), ())),
                            precision=_HIGH,
                            preferred_element_type=jnp.float32)
    rows = jnp.where(denom > 0.0, numer / denom, 0.0)
    remap = lax.dot_general(sel_ref[...], rows, (((1,), (0,)), ((), ())),
                            precision=_HIGH,
                            preferred_element_type=jnp.float32)
    out_ref[...] = lax.dot_general(
        remap, wh_ref[...], (((1,), (0,)), ((), ())),
        precision=_HIGH, preferred_element_type=jnp.float32) + bh_ref[...]


_finish_call = pl.pallas_call(
    _finish_body,
    out_shape=jax.ShapeDtypeStruct((G, 128), jnp.float32),
)


@jax.jit
def kernel(x, edge_index, batch, W_emb, b_emb, W_a1, b_a1, W_a2, b_a2,
           W_head, b_head):
    f32 = jnp.float32
    bp = jnp.full((1, NP), G, jnp.int32).at[0, :N].set(batch)
    num_nodes, inv_row, sel = _stats_call(bp)

    xp = jnp.zeros((NP, x.shape[1]), f32).at[:N].set(x)
    emb, att_row = _dense_call(xp, W_emb, b_emb.reshape(1, -1),
                               W_a1, b_a1.reshape(1, -1),
                               W_a2, b_a2.reshape(1, 1))

    src = edge_index[0]
    dst = edge_index[1]
    srcp = jnp.concatenate([src, jnp.zeros((EPAD - E,), src.dtype)])
    dstp = jnp.concatenate([dst, jnp.full((EPAD - E,), NP - 1, dst.dtype)])
    zero_c = jnp.zeros((CLEN,), f32)
    cparts = _sc_counts_call(srcp, dstp, inv_row.reshape(NP), zero_c)
    cparts = cparts.reshape(2, NSLOT, NP)

    out = _finish_call(cparts[0, :G], cparts[1, :G], att_row, emb, sel,
                       W_head, b_head.reshape(1, -1))
    return (out, num_nodes)


# speedup vs baseline: 98.8707x; 1.7705x over previous
import functools

import jax
import jax.numpy as jnp
from jax import lax
from jax.experimental import pallas as pl
from jax.experimental.pallas import tpu as pltpu
from jax.experimental.pallas import tpu_sc as plsc

N = 10000
NP = 10240
G = 64
TRASH = G
NSLOT = G + 1
CLEN = NSLOT * NP

E = 320000
CH = 128
NWORK = 32
EW = 10112
EPAD = NWORK * EW
STAGE = EW + 2 * CH

_HIGH = jax.lax.Precision.HIGHEST


def _leaky(v):
    return jax.nn.leaky_relu(v, negative_slope=0.01)


def _stats_body(b_ref, nn_ref, inv_ref, sel_ref):
    bp = b_ref[...]
    gcol = lax.broadcasted_iota(jnp.int32, (G, 1), 0)
    eqg = bp == gcol
    counts = jnp.sum(eqg.astype(jnp.float32), axis=1, keepdims=True)
    nn_ref[...] = counts
    first_pos = jnp.sum((bp < gcol).astype(jnp.int32), axis=1, keepdims=True)
    tgt = jnp.where(counts > 0.0, first_pos, N - 1)
    ids = lax.broadcasted_iota(jnp.int32, (1, NP), 1)
    match = ids == tgt
    winner = jnp.max(jnp.where(match, gcol, -1), axis=0, keepdims=True)
    inv_ref[...] = jnp.where(winner >= 0, winner, TRASH)
    tgtf = tgt.astype(jnp.float32)
    eye = (lax.broadcasted_iota(jnp.int32, (G, G), 0)
           == lax.broadcasted_iota(jnp.int32, (G, G), 1)).astype(jnp.float32)
    tgt_row = lax.dot_general(tgtf, eye, (((0,), (0,)), ((), ())),
                              precision=_HIGH,
                              preferred_element_type=jnp.float32)
    mm = tgtf == tgt_row
    wio = lax.broadcasted_iota(jnp.int32, (1, G), 1)
    owner = jnp.max(jnp.where(mm, wio, -1), axis=1, keepdims=True)
    sel_ref[...] = (owner == wio).astype(jnp.float32)


_stats_call = pl.pallas_call(
    _stats_body,
    out_shape=(
        jax.ShapeDtypeStruct((G, 1), jnp.float32),
        jax.ShapeDtypeStruct((1, NP), jnp.int32),
        jax.ShapeDtypeStruct((G, G), jnp.float32),
    ),
)


def _dense_body(x_ref, we_ref, be_ref, wa1_ref, ba1_ref, wa2_ref, ba2_ref,
                emb_ref, att_ref):
    x = x_ref[...]
    emb_ref[...] = _leaky(
        lax.dot_general(x, we_ref[...], (((1,), (0,)), ((), ())),
                        precision=_HIGH, preferred_element_type=jnp.float32)
        + be_ref[...])
    a1 = _leaky(
        lax.dot_general(x, wa1_ref[...], (((1,), (0,)), ((), ())),
                        precision=_HIGH, preferred_element_type=jnp.float32)
        + ba1_ref[...])
    att_ref[...] = lax.dot_general(
        wa2_ref[...], a1, (((0,), (1,)), ((), ())),
        precision=_HIGH, preferred_element_type=jnp.float32) + ba2_ref[...]


_dense_call = pl.pallas_call(
    _dense_body,
    out_shape=(
        jax.ShapeDtypeStruct((NP, 128), jnp.float32),
        jax.ShapeDtypeStruct((1, NP), jnp.float32),
    ),
)


def _sc_counts_body(src_hbm, dst_hbm, inv_hbm, zero_hbm, out_hbm,
                    inv_v, src_v, dst_v, stage_v, idx_v, ones_v, csh):
    cid = lax.axis_index("c")
    sid = lax.axis_index("s")
    wid = sid * 2 + cid
    zch = CLEN // 16

    pltpu.sync_copy(zero_hbm.at[pl.ds(sid * zch, zch)],
                    csh.at[pl.ds(sid * zch, zch)])
    pltpu.sync_copy(inv_hbm, inv_v)
    base = wid * EW
    pltpu.sync_copy(src_hbm.at[pl.ds(base, EW)], src_v)
    pltpu.sync_copy(dst_hbm.at[pl.ds(base, EW)], dst_v)
    for i in range(CH // 16):
        ones_v[pl.ds(i * 16, 16)] = jnp.full((16,), 1.0, jnp.float32)
    trash16 = jnp.full((16,), TRASH * NP, jnp.int32)

    @pl.loop(0, STAGE // 16)
    def _fill(i):
        stage_v[pl.ds(i * 16, 16)] = trash16

    @pl.loop(0, EW // 16, init_carry=jnp.int32(0))
    def _scan(i, cur):
        sl = pl.ds(i * 16, 16)
        d16 = dst_v[sl]
        s16 = src_v[sl]
        slot16 = plsc.load_gather(inv_v, [d16])
        keep = slot16 < TRASH
        plsc.store_compressed(stage_v.at[pl.ds(cur, 16)],
                              slot16 * NP + s16, mask=keep)
        cnt = plsc.all_reduce_population_count(keep)
        return cur + jnp.max(cnt)

    cur = _scan
    nchunk = (cur + CH - 1) // CH
    plsc.subcore_barrier()

    @pl.loop(0, 16)
    def _turn(t):
        @pl.when(sid == t)
        def _my_turn():
            @pl.loop(0, nchunk)
            def _chunk(j):
                for i in range(CH // 16):
                    idx_v[0, pl.ds(i * 16, 16)] = (
                        stage_v[pl.ds(j * CH + i * 16, 16)])
                pltpu.sync_copy(ones_v, csh.at[idx_v.at[0]], add=True)

        plsc.subcore_barrier()

    pltpu.sync_copy(csh.at[pl.ds(sid * zch, zch)],
                    out_hbm.at[cid, pl.ds(sid * zch, zch)])


_sc_counts_call = pl.kernel(
    _sc_counts_body,
    out_type=jax.ShapeDtypeStruct((2, CLEN), jnp.float32),
    mesh=plsc.VectorSubcoreMesh(core_axis_name="c", subcore_axis_name="s"),
    compiler_params=pltpu.CompilerParams(needs_layout_passes=False),
    scratch_types=[
        pltpu.VMEM((NP,), jnp.int32),
        pltpu.VMEM((EW,), jnp.int32),
        pltpu.VMEM((EW,), jnp.int32),
        pltpu.VMEM((STAGE,), jnp.int32),
        pltpu.VMEM((1, CH), jnp.int32),
        pltpu.VMEM((CH,), jnp.float32),
        pltpu.VMEM_SHARED((CLEN,), jnp.float32),
    ],
)


def _finish_body(c0_ref, c1_ref, att_ref, emb_ref, sel_ref, wh_ref, bh_ref,
                 out_ref):
    cs = c0_ref[...] + c1_ref[...]
    att = att_ref[...]
    pos = cs > 0.0
    m = jnp.max(jnp.where(pos, att, -jnp.inf), axis=1, keepdims=True)
    p = jnp.where(pos, cs * jnp.exp(att - m), 0.0)
    denom = jnp.sum(p, axis=1, keepdims=True)
    numer = lax.dot_general(p, emb_ref[...], (((1,), (0,)), ((), ())),
                            precision=_HIGH,
                            preferred_element_type=jnp.float32)
    rows = jnp.where(denom > 0.0, numer / denom, 0.0)
    remap = lax.dot_general(sel_ref[...], rows, (((1,), (0,)), ((), ())),
                            precision=_HIGH,
                            preferred_element_type=jnp.float32)
    out_ref[...] = lax.dot_general(
        remap, wh_ref[...], (((1,), (0,)), ((), ())),
        precision=_HIGH, preferred_element_type=jnp.float32) + bh_ref[...]


_finish_call = pl.pallas_call(
    _finish_body,
    out_shape=jax.ShapeDtypeStruct((G, 128), jnp.float32),
)


@jax.jit
def kernel(x, edge_index, batch, W_emb, b_emb, W_a1, b_a1, W_a2, b_a2,
           W_head, b_head):
    f32 = jnp.float32
    bp = jnp.full((1, NP), G, jnp.int32).at[0, :N].set(batch)
    num_nodes, inv_row, sel = _stats_call(bp)

    xp = jnp.zeros((NP, x.shape[1]), f32).at[:N].set(x)
    emb, att_row = _dense_call(xp, W_emb, b_emb.reshape(1, -1),
                               W_a1, b_a1.reshape(1, -1),
                               W_a2, b_a2.reshape(1, 1))

    src = edge_index[0]
    dst = edge_index[1]
    srcp = jnp.concatenate([src, jnp.zeros((EPAD - E,), src.dtype)])
    dstp = jnp.concatenate([dst, jnp.full((EPAD - E,), NP - 1, dst.dtype)])
    zero_c = jnp.zeros((CLEN,), f32)
    cparts = _sc_counts_call(srcp, dstp, inv_row.reshape(NP), zero_c)
    cparts = cparts.reshape(2, NSLOT, NP)

    out = _finish_call(cparts[0, :G], cparts[1, :G], att_row, emb, sel,
                       W_head, b_head.reshape(1, -1))
    return (out, num_nodes)
